# baseline (speedup 1.0000x reference)
import jax
import jax.numpy as jnp
from jax.experimental import pallas as pl

_G = 12
_B = 349568
_BD = 350208


def _copy_body(x_ref, r_ref, d_ref, xo_ref, ro_ref, do_ref):
    xo_ref[...] = x_ref[...]
    ro_ref[...] = r_ref[...]
    do_ref[...] = d_ref[...]


def kernel(sampled_point_xyz, sampled_point_ray_direction, sampled_point_distance):
    n = sampled_point_xyz.shape[0]
    xt = sampled_point_xyz.T
    rt = sampled_point_ray_direction.T
    pos_t, ray_t, dists = pl.pallas_call(
        _copy_body,
        grid=(_G,),
        in_specs=[
            pl.BlockSpec((3, _B), lambda i: (0, i)),
            pl.BlockSpec((3, _B), lambda i: (0, i)),
            pl.BlockSpec((_BD,), lambda i: (i,)),
        ],
        out_specs=[
            pl.BlockSpec((3, _B), lambda i: (0, i)),
            pl.BlockSpec((3, _B), lambda i: (0, i)),
            pl.BlockSpec((_BD,), lambda i: (i,)),
        ],
        out_shape=[
            jax.ShapeDtypeStruct((3, n), jnp.float32),
            jax.ShapeDtypeStruct((3, n), jnp.float32),
            jax.ShapeDtypeStruct((n,), jnp.float32),
        ],
    )(xt, rt, sampled_point_distance)
    return (pos_t.T, ray_t.T, dists)
